# baseline (speedup 1.0000x reference)
import jax
import jax.numpy as jnp
from jax import lax
from jax.experimental import pallas as pl
from jax.experimental.pallas import tpu as pltpu
from jax.experimental.pallas import tpu_sc as plsc

NUM_HEADS = 16
T_STATIC = 2048
NUM_BUCKETS = 257
LANES = 16
NUM_COLS = 23
R_MINOR = NUM_COLS * 128
OUT_SHAPE = (1, NUM_HEADS, T_STATIC, T_STATIC)


def _sc_body(table_hbm, out_hbm):
    def inner(table_v, r_v, sem):
        c = lax.axis_index("c")
        s = lax.axis_index("s")
        h = s
        a0 = c * 128

        pltpu.sync_copy(table_hbm, table_v)

        hvec = jnp.full((LANES,), h, dtype=jnp.int32)
        iot = lax.iota(jnp.int32, LANES)

        def fill_column(jj, t):
            col0 = jj * 128
            for rr in range(8):
                for u in range(8):
                    ll = u * LANES + iot
                    bucket = jnp.clip(8 * t + rr - ll, -128, 128) + 128
                    val = plsc.load_gather(table_v, [bucket, hvec])
                    off = pl.multiple_of(col0 + u * LANES, LANES)
                    r_v[rr, pl.ds(off, LANES)] = val

        def init(jj, _):
            fill_column(jj, a0 + 112 - 16 * jj)
            return _

        lax.fori_loop(0, NUM_COLS, init, None)

        def emit(r, _):
            for dcol in range(4):
                jj = c * 8 + 5 + dcol
                t = r + 32 - 16 * dcol
                fill_column(jj, t)
            descs = []
            for k in range(8):
                a = a0 + r + 16 * k
                b0 = jnp.clip(c * 8 + k - 1, 0, 12)
                jj0 = 7 - k + b0
                src = r_v.at[:, pl.ds(pl.multiple_of(128 * jj0, 128), 512)]
                dst = out_hbm.at[
                    0,
                    h,
                    pl.ds(pl.multiple_of(8 * a, 8), 8),
                    pl.ds(pl.multiple_of(128 * b0, 128), 512),
                ]
                descs.append(pltpu.async_copy(src, dst, sem))
            for d in descs:
                d.wait()
            return _

        lax.fori_loop(0, 16, emit, None)

    pl.run_scoped(
        inner,
        pltpu.VMEM((NUM_BUCKETS, NUM_HEADS), jnp.float32),
        pltpu.VMEM((8, R_MINOR), jnp.float32),
        pltpu.SemaphoreType.DMA,
    )


def _tc_body(table_ref, band_ref, out_ref, smem_ref, b256, b0v, sems):
    del band_ref
    d1 = pltpu.make_async_copy(
        table_ref.at[pl.ds(0, 1)], smem_ref.at[pl.ds(0, 1)], sems.at[0]
    )
    d2 = pltpu.make_async_copy(
        table_ref.at[pl.ds(256, 1)], smem_ref.at[pl.ds(1, 1)], sems.at[1]
    )
    d1.start()
    d2.start()
    d1.wait()
    d2.wait()

    pending = {0: [], 1: []}
    for h in range(NUM_HEADS):
        p = h % 2
        for d in pending[p]:
            d.wait()
        pending[p] = []
        v256 = smem_ref[1, h]
        v0 = smem_ref[0, h]
        b256[p, ...] = jnp.full((128, T_STATIC), v256, jnp.float32)
        b0v[p, ...] = jnp.full((128, T_STATIC), v0, jnp.float32)
        for g in range(16):
            rows = pl.ds(128 * g, 128)
            wl = 128 * (g - 1)
            if wl > 0:
                d = pltpu.make_async_copy(
                    b256.at[p, :, pl.ds(0, wl)],
                    out_ref.at[0, h, rows, pl.ds(0, wl)],
                    sems.at[p],
                )
                d.start()
                pending[p].append(d)
            wr = 128 * (13 - g)
            if wr > 0:
                d = pltpu.make_async_copy(
                    b0v.at[p, :, pl.ds(0, wr)],
                    out_ref.at[0, h, rows, pl.ds(128 * (g + 3), wr)],
                    sems.at[p],
                )
                d.start()
                pending[p].append(d)
    for p in (0, 1):
        for d in pending[p]:
            d.wait()


@jax.jit
def _run(bias_table):
    sc_mesh = plsc.VectorSubcoreMesh(
        core_axis_name="c", subcore_axis_name="s", num_cores=2, num_subcores=16
    )
    sc_band = pl.kernel(
        _sc_body,
        out_type=jax.ShapeDtypeStruct(OUT_SHAPE, jnp.float32),
        mesh=sc_mesh,
        compiler_params=pltpu.CompilerParams(needs_layout_passes=False),
    )
    band = sc_band(bias_table)

    tc_const = pl.pallas_call(
        _tc_body,
        out_shape=jax.ShapeDtypeStruct(OUT_SHAPE, jnp.float32),
        in_specs=[
            pl.BlockSpec(memory_space=pl.ANY),
            pl.BlockSpec(memory_space=pl.ANY),
        ],
        out_specs=pl.BlockSpec(memory_space=pl.ANY),
        scratch_shapes=[
            pltpu.SMEM((2, NUM_HEADS), jnp.float32),
            pltpu.VMEM((2, 128, T_STATIC), jnp.float32),
            pltpu.VMEM((2, 128, T_STATIC), jnp.float32),
            pltpu.SemaphoreType.DMA((2,)),
        ],
        input_output_aliases={1: 0},
    )
    return tc_const(bias_table, band)


def kernel(T, bias_table):
    return _run(bias_table)


# speedup vs baseline: 1.1866x; 1.1866x over previous
import jax
import jax.numpy as jnp
from jax import lax
from jax.experimental import pallas as pl
from jax.experimental.pallas import tpu as pltpu
from jax.experimental.pallas import tpu_sc as plsc

NUM_HEADS = 16
T_STATIC = 2048
NUM_BUCKETS = 257
LANES = 16
B_COLS = 5
B_MINOR = 16 * B_COLS * 128
OUT_SHAPE = (1, NUM_HEADS, T_STATIC, T_STATIC)


def _sc_body(table_hbm, out_hbm):
    def inner(table_v, b_v, ct_v, sem):
        c = lax.axis_index("c")
        s = lax.axis_index("s")
        h = s
        a0 = c * 128

        pltpu.sync_copy(table_hbm, table_v)

        hvec = jnp.full((LANES,), h, dtype=jnp.int32)
        iot = lax.iota(jnp.int32, LANES)

        v256 = plsc.load_gather(table_v, [jnp.full((LANES,), 256, jnp.int32), hvec])
        v0 = plsc.load_gather(table_v, [jnp.full((LANES,), 0, jnp.int32), hvec])

        for rr in range(8):
            for u in range(8):
                ct_v[rr, pl.ds(u * LANES, LANES)] = v256
                ct_v[rr, pl.ds(128 + u * LANES, LANES)] = v0

        def build(r):
            base = r * (B_COLS * 128)
            for j in range(B_COLS):
                t = r + 32 - 16 * j
                col0 = base + j * 128
                for rr in range(8):
                    for u in range(8):
                        off = pl.multiple_of(col0 + u * LANES, LANES)
                        if j == 0:
                            b_v[rr, pl.ds(off, LANES)] = v256
                        elif j == B_COLS - 1:
                            b_v[rr, pl.ds(off, LANES)] = v0
                        else:
                            ll = u * LANES + iot
                            bucket = jnp.clip(8 * t + rr - ll, -128, 128) + 128
                            val = plsc.load_gather(table_v, [bucket, hvec])
                            b_v[rr, pl.ds(off, LANES)] = val

        build(0)

        def emit(r, _):
            base = r * (B_COLS * 128)
            descs = []
            for k in range(8):
                a = a0 + r + 16 * k
                rows = pl.ds(pl.multiple_of(8 * a, 8), 8)
                edge_lo = c == 0 if k == 0 else None
                edge_hi = c == 1 if k == 7 else None
                if edge_lo is None and edge_hi is None:
                    b0 = c * 8 + k - 1
                    src = b_v.at[:, pl.ds(pl.multiple_of(base + 128, 128), 512)]
                    dst = out_hbm.at[
                        0, h, rows, pl.ds(pl.multiple_of(128 * b0, 128), 512)
                    ]
                    descs.append(pltpu.async_copy(src, dst, sem))
                else:
                    cond = edge_lo if edge_lo is not None else edge_hi

                    @pl.when(jnp.logical_not(cond))
                    def _():
                        b0 = c * 8 + k - 1
                        src = b_v.at[
                            :, pl.ds(pl.multiple_of(base + 128, 128), 512)
                        ]
                        dst = out_hbm.at[
                            0, h, rows, pl.ds(pl.multiple_of(128 * b0, 128), 512)
                        ]
                        pltpu.async_copy(src, dst, sem)

                    if k == 0:
                        @pl.when(cond)
                        def _():
                            src = b_v.at[
                                :, pl.ds(pl.multiple_of(base + 256, 128), 384)
                            ]
                            dst = out_hbm.at[0, h, rows, pl.ds(0, 384)]
                            pltpu.async_copy(src, dst, sem)
                            src2 = ct_v.at[:, pl.ds(128, 128)]
                            dst2 = out_hbm.at[0, h, rows, pl.ds(384, 128)]
                            pltpu.async_copy(src2, dst2, sem)

                    else:
                        @pl.when(cond)
                        def _():
                            src2 = ct_v.at[:, pl.ds(0, 128)]
                            dst2 = out_hbm.at[0, h, rows, pl.ds(1536, 128)]
                            pltpu.async_copy(src2, dst2, sem)
                            src = b_v.at[:, pl.ds(pl.multiple_of(base, 128), 384)]
                            dst = out_hbm.at[0, h, rows, pl.ds(1664, 384)]
                            pltpu.async_copy(src, dst, sem)

                    descs.append(None)

            @pl.when(r < 15)
            def _():
                build(r + 1)

            drain = pltpu.make_async_copy(
                b_v.at[:, pl.ds(0, 512)],
                out_hbm.at[0, h, pl.ds(8 * a0, 8), pl.ds(0, 512)],
                sem,
            )
            for _i in range(8):
                drain.wait()
            return _

        lax.fori_loop(0, 16, emit, None)

    pl.run_scoped(
        inner,
        pltpu.VMEM((NUM_BUCKETS, NUM_HEADS), jnp.float32),
        pltpu.VMEM((8, B_MINOR), jnp.float32),
        pltpu.VMEM((8, 256), jnp.float32),
        pltpu.SemaphoreType.DMA,
    )


def _tc_body(table_ref, band_ref, out_ref, smem_ref, b256, b0v, sems):
    del band_ref
    d1 = pltpu.make_async_copy(
        table_ref.at[pl.ds(0, 1)], smem_ref.at[pl.ds(0, 1)], sems.at[0]
    )
    d2 = pltpu.make_async_copy(
        table_ref.at[pl.ds(256, 1)], smem_ref.at[pl.ds(1, 1)], sems.at[1]
    )
    d1.start()
    d2.start()
    d1.wait()
    d2.wait()

    pending = {0: [], 1: []}
    for h in range(NUM_HEADS):
        p = h % 2
        for d in pending[p]:
            d.wait()
        pending[p] = []
        v256 = smem_ref[1, h]
        v0 = smem_ref[0, h]
        b256[p, ...] = jnp.full((128, T_STATIC), v256, jnp.float32)
        b0v[p, ...] = jnp.full((128, T_STATIC), v0, jnp.float32)
        for g in range(16):
            rows = pl.ds(128 * g, 128)
            wl = 128 * (g - 1)
            if wl > 0:
                d = pltpu.make_async_copy(
                    b256.at[p, :, pl.ds(0, wl)],
                    out_ref.at[0, h, rows, pl.ds(0, wl)],
                    sems.at[p],
                )
                d.start()
                pending[p].append(d)
            wr = 128 * (13 - g)
            if wr > 0:
                d = pltpu.make_async_copy(
                    b0v.at[p, :, pl.ds(0, wr)],
                    out_ref.at[0, h, rows, pl.ds(128 * (g + 3), wr)],
                    sems.at[p],
                )
                d.start()
                pending[p].append(d)
    for p in (0, 1):
        for d in pending[p]:
            d.wait()


@jax.jit
def _run(bias_table):
    sc_mesh = plsc.VectorSubcoreMesh(
        core_axis_name="c", subcore_axis_name="s", num_cores=2, num_subcores=16
    )
    sc_band = pl.kernel(
        _sc_body,
        out_type=jax.ShapeDtypeStruct(OUT_SHAPE, jnp.float32),
        mesh=sc_mesh,
        compiler_params=pltpu.CompilerParams(needs_layout_passes=False),
    )
    band = sc_band(bias_table)

    tc_const = pl.pallas_call(
        _tc_body,
        out_shape=jax.ShapeDtypeStruct(OUT_SHAPE, jnp.float32),
        in_specs=[
            pl.BlockSpec(memory_space=pl.ANY),
            pl.BlockSpec(memory_space=pl.ANY),
        ],
        out_specs=pl.BlockSpec(memory_space=pl.ANY),
        scratch_shapes=[
            pltpu.SMEM((2, NUM_HEADS), jnp.float32),
            pltpu.VMEM((2, 128, T_STATIC), jnp.float32),
            pltpu.VMEM((2, 128, T_STATIC), jnp.float32),
            pltpu.SemaphoreType.DMA((2,)),
        ],
        input_output_aliases={1: 0},
    )
    return tc_const(bias_table, band)


def kernel(T, bias_table):
    return _run(bias_table)


# speedup vs baseline: 1.1953x; 1.0074x over previous
import jax
import jax.numpy as jnp
from jax import lax
from jax.experimental import pallas as pl
from jax.experimental.pallas import tpu as pltpu
from jax.experimental.pallas import tpu_sc as plsc

NUM_HEADS = 16
T_STATIC = 2048
NUM_BUCKETS = 257
LANES = 16
B_COLS = 5
B_MINOR = 16 * B_COLS * 128
OUT_SHAPE = (1, NUM_HEADS, T_STATIC, T_STATIC)


def _sc_body(table_hbm, out_hbm):
    def inner(table_v, b_v, ct_v, sem):
        c = lax.axis_index("c")
        s = lax.axis_index("s")
        h = s
        a0 = c * 128

        pltpu.sync_copy(table_hbm, table_v)

        hvec = jnp.full((LANES,), h, dtype=jnp.int32)
        iot = lax.iota(jnp.int32, LANES)

        v256 = plsc.load_gather(table_v, [jnp.full((LANES,), 256, jnp.int32), hvec])
        v0 = plsc.load_gather(table_v, [jnp.full((LANES,), 0, jnp.int32), hvec])

        for rr in range(8):
            for u in range(8):
                ct_v[rr, pl.ds(u * LANES, LANES)] = v256
                ct_v[rr, pl.ds(128 + u * LANES, LANES)] = v0

        def build(r):
            base = r * (B_COLS * 128)
            for j in range(B_COLS):
                t = r + 32 - 16 * j
                col0 = base + j * 128
                for rr in range(8):
                    for u in range(8):
                        off = pl.multiple_of(col0 + u * LANES, LANES)
                        if j == 0:
                            b_v[rr, pl.ds(off, LANES)] = v256
                        elif j == B_COLS - 1:
                            b_v[rr, pl.ds(off, LANES)] = v0
                        else:
                            ll = u * LANES + iot
                            bucket = jnp.clip(8 * t + rr - ll, -128, 128) + 128
                            val = plsc.load_gather(table_v, [bucket, hvec])
                            b_v[rr, pl.ds(off, LANES)] = val

        build(0)

        def emit(r, _):
            base = r * (B_COLS * 128)
            descs = []
            for k in range(8):
                a = a0 + r + 16 * k
                rows = pl.ds(pl.multiple_of(8 * a, 8), 8)
                edge_lo = c == 0 if k == 0 else None
                edge_hi = c == 1 if k == 7 else None
                if edge_lo is None and edge_hi is None:
                    b0 = c * 8 + k - 1
                    src = b_v.at[:, pl.ds(pl.multiple_of(base + 128, 128), 512)]
                    dst = out_hbm.at[
                        0, h, rows, pl.ds(pl.multiple_of(128 * b0, 128), 512)
                    ]
                    descs.append(pltpu.async_copy(src, dst, sem))
                else:
                    cond = edge_lo if edge_lo is not None else edge_hi

                    @pl.when(jnp.logical_not(cond))
                    def _():
                        b0 = c * 8 + k - 1
                        src = b_v.at[
                            :, pl.ds(pl.multiple_of(base + 128, 128), 512)
                        ]
                        dst = out_hbm.at[
                            0, h, rows, pl.ds(pl.multiple_of(128 * b0, 128), 512)
                        ]
                        pltpu.async_copy(src, dst, sem)

                    if k == 0:
                        @pl.when(cond)
                        def _():
                            src = b_v.at[
                                :, pl.ds(pl.multiple_of(base + 256, 128), 384)
                            ]
                            dst = out_hbm.at[0, h, rows, pl.ds(0, 384)]
                            pltpu.async_copy(src, dst, sem)
                            src2 = ct_v.at[:, pl.ds(128, 128)]
                            dst2 = out_hbm.at[0, h, rows, pl.ds(384, 128)]
                            pltpu.async_copy(src2, dst2, sem)

                    else:
                        @pl.when(cond)
                        def _():
                            src2 = ct_v.at[:, pl.ds(0, 128)]
                            dst2 = out_hbm.at[0, h, rows, pl.ds(1536, 128)]
                            pltpu.async_copy(src2, dst2, sem)
                            src = b_v.at[:, pl.ds(pl.multiple_of(base, 128), 384)]
                            dst = out_hbm.at[0, h, rows, pl.ds(1664, 384)]
                            pltpu.async_copy(src, dst, sem)

                    descs.append(None)

            @pl.when(r < 15)
            def _():
                build(r + 1)

            @pl.when(r > 0)
            def _():
                drain = pltpu.make_async_copy(
                    b_v.at[:, pl.ds(0, 512)],
                    out_hbm.at[0, h, pl.ds(8 * a0, 8), pl.ds(0, 512)],
                    sem,
                )
                for _i in range(8):
                    drain.wait()

            return _

        lax.fori_loop(0, 16, emit, None)

        final = pltpu.make_async_copy(
            b_v.at[:, pl.ds(0, 512)],
            out_hbm.at[0, h, pl.ds(8 * a0, 8), pl.ds(0, 512)],
            sem,
        )
        for _i in range(8):
            final.wait()

    pl.run_scoped(
        inner,
        pltpu.VMEM((NUM_BUCKETS, NUM_HEADS), jnp.float32),
        pltpu.VMEM((8, B_MINOR), jnp.float32),
        pltpu.VMEM((8, 256), jnp.float32),
        pltpu.SemaphoreType.DMA,
    )


def _tc_body(table_ref, band_ref, out_ref, smem_ref, b256, b0v, sems):
    del band_ref
    d1 = pltpu.make_async_copy(
        table_ref.at[pl.ds(0, 1)], smem_ref.at[pl.ds(0, 1)], sems.at[0]
    )
    d2 = pltpu.make_async_copy(
        table_ref.at[pl.ds(256, 1)], smem_ref.at[pl.ds(1, 1)], sems.at[1]
    )
    d1.start()
    d2.start()
    d1.wait()
    d2.wait()

    pending = {0: [], 1: []}
    for h in range(NUM_HEADS):
        p = h % 2
        for d in pending[p]:
            d.wait()
        pending[p] = []
        v256 = smem_ref[1, h]
        v0 = smem_ref[0, h]
        b256[p, ...] = jnp.full((128, T_STATIC), v256, jnp.float32)
        b0v[p, ...] = jnp.full((128, T_STATIC), v0, jnp.float32)
        for g in range(16):
            rows = pl.ds(128 * g, 128)
            wl = 128 * (g - 1)
            if wl > 0:
                d = pltpu.make_async_copy(
                    b256.at[p, :, pl.ds(0, wl)],
                    out_ref.at[0, h, rows, pl.ds(0, wl)],
                    sems.at[p],
                )
                d.start()
                pending[p].append(d)
            wr = 128 * (13 - g)
            if wr > 0:
                d = pltpu.make_async_copy(
                    b0v.at[p, :, pl.ds(0, wr)],
                    out_ref.at[0, h, rows, pl.ds(128 * (g + 3), wr)],
                    sems.at[p],
                )
                d.start()
                pending[p].append(d)
    for p in (0, 1):
        for d in pending[p]:
            d.wait()


@jax.jit
def _run(bias_table):
    sc_mesh = plsc.VectorSubcoreMesh(
        core_axis_name="c", subcore_axis_name="s", num_cores=2, num_subcores=16
    )
    sc_band = pl.kernel(
        _sc_body,
        out_type=jax.ShapeDtypeStruct(OUT_SHAPE, jnp.float32),
        mesh=sc_mesh,
        compiler_params=pltpu.CompilerParams(needs_layout_passes=False),
    )
    band = sc_band(bias_table)

    tc_const = pl.pallas_call(
        _tc_body,
        out_shape=jax.ShapeDtypeStruct(OUT_SHAPE, jnp.float32),
        in_specs=[
            pl.BlockSpec(memory_space=pl.ANY),
            pl.BlockSpec(memory_space=pl.ANY),
        ],
        out_specs=pl.BlockSpec(memory_space=pl.ANY),
        scratch_shapes=[
            pltpu.SMEM((2, NUM_HEADS), jnp.float32),
            pltpu.VMEM((2, 128, T_STATIC), jnp.float32),
            pltpu.VMEM((2, 128, T_STATIC), jnp.float32),
            pltpu.SemaphoreType.DMA((2,)),
        ],
        input_output_aliases={1: 0},
    )
    return tc_const(bias_table, band)


def kernel(T, bias_table):
    return _run(bias_table)


# speedup vs baseline: 1.2129x; 1.0147x over previous
import jax
import jax.numpy as jnp
from jax import lax
from jax.experimental import pallas as pl
from jax.experimental.pallas import tpu as pltpu
from jax.experimental.pallas import tpu_sc as plsc
from jax._src.pallas import mpmd as _mpmd

NUM_HEADS = 16
T_STATIC = 2048
NUM_BUCKETS = 257
LANES = 16
B_COLS = 5
B_MINOR = 16 * B_COLS * 128
OUT_SHAPE = (1, NUM_HEADS, T_STATIC, T_STATIC)


def _sc_body(table_hbm, tcbuf_hbm, out_hbm):
    del tcbuf_hbm

    def inner(table_v, b_v, ct_v, sem):
        c = lax.axis_index("c")
        s = lax.axis_index("s")
        h = s
        a0 = c * 128

        pltpu.sync_copy(table_hbm, table_v)

        hvec = jnp.full((LANES,), h, dtype=jnp.int32)
        iot = lax.iota(jnp.int32, LANES)

        v256 = plsc.load_gather(table_v, [jnp.full((LANES,), 256, jnp.int32), hvec])
        v0 = plsc.load_gather(table_v, [jnp.full((LANES,), 0, jnp.int32), hvec])

        for rr in range(8):
            for u in range(8):
                ct_v[rr, pl.ds(u * LANES, LANES)] = v256
                ct_v[rr, pl.ds(128 + u * LANES, LANES)] = v0

        def build(r):
            base = r * (B_COLS * 128)
            for j in range(B_COLS):
                t = r + 32 - 16 * j
                col0 = base + j * 128
                for rr in range(8):
                    for u in range(8):
                        off = pl.multiple_of(col0 + u * LANES, LANES)
                        if j == 0:
                            b_v[rr, pl.ds(off, LANES)] = v256
                        elif j == B_COLS - 1:
                            b_v[rr, pl.ds(off, LANES)] = v0
                        else:
                            ll = u * LANES + iot
                            bucket = jnp.clip(8 * t + rr - ll, -128, 128) + 128
                            val = plsc.load_gather(table_v, [bucket, hvec])
                            b_v[rr, pl.ds(off, LANES)] = val

        build(0)

        def emit(r, _):
            base = r * (B_COLS * 128)
            descs = []
            for k in range(8):
                a = a0 + r + 16 * k
                rows = pl.ds(pl.multiple_of(8 * a, 8), 8)
                edge_lo = c == 0 if k == 0 else None
                edge_hi = c == 1 if k == 7 else None
                if edge_lo is None and edge_hi is None:
                    b0 = c * 8 + k - 1
                    src = b_v.at[:, pl.ds(pl.multiple_of(base + 128, 128), 512)]
                    dst = out_hbm.at[
                        0, h, rows, pl.ds(pl.multiple_of(128 * b0, 128), 512)
                    ]
                    descs.append(pltpu.async_copy(src, dst, sem))
                else:
                    cond = edge_lo if edge_lo is not None else edge_hi

                    @pl.when(jnp.logical_not(cond))
                    def _():
                        b0 = c * 8 + k - 1
                        src = b_v.at[
                            :, pl.ds(pl.multiple_of(base + 128, 128), 512)
                        ]
                        dst = out_hbm.at[
                            0, h, rows, pl.ds(pl.multiple_of(128 * b0, 128), 512)
                        ]
                        pltpu.async_copy(src, dst, sem)

                    if k == 0:
                        @pl.when(cond)
                        def _():
                            src = b_v.at[
                                :, pl.ds(pl.multiple_of(base + 256, 128), 384)
                            ]
                            dst = out_hbm.at[0, h, rows, pl.ds(0, 384)]
                            pltpu.async_copy(src, dst, sem)
                            src2 = ct_v.at[:, pl.ds(128, 128)]
                            dst2 = out_hbm.at[0, h, rows, pl.ds(384, 128)]
                            pltpu.async_copy(src2, dst2, sem)

                    else:
                        @pl.when(cond)
                        def _():
                            src2 = ct_v.at[:, pl.ds(0, 128)]
                            dst2 = out_hbm.at[0, h, rows, pl.ds(1536, 128)]
                            pltpu.async_copy(src2, dst2, sem)
                            src = b_v.at[:, pl.ds(pl.multiple_of(base, 128), 384)]
                            dst = out_hbm.at[0, h, rows, pl.ds(1664, 384)]
                            pltpu.async_copy(src, dst, sem)

                    descs.append(None)

            @pl.when(r < 15)
            def _():
                build(r + 1)

            @pl.when(r > 0)
            def _():
                drain = pltpu.make_async_copy(
                    b_v.at[:, pl.ds(0, 512)],
                    out_hbm.at[0, h, pl.ds(8 * a0, 8), pl.ds(0, 512)],
                    sem,
                )
                for _i in range(8):
                    drain.wait()

            return _

        lax.fori_loop(0, 16, emit, None)

        final = pltpu.make_async_copy(
            b_v.at[:, pl.ds(0, 512)],
            out_hbm.at[0, h, pl.ds(8 * a0, 8), pl.ds(0, 512)],
            sem,
        )
        for _i in range(8):
            final.wait()

    pl.run_scoped(
        inner,
        pltpu.VMEM((NUM_BUCKETS, NUM_HEADS), jnp.float32),
        pltpu.VMEM((8, B_MINOR), jnp.float32),
        pltpu.VMEM((8, 256), jnp.float32),
        pltpu.SemaphoreType.DMA,
    )


def _tc_body(table_ref, out_ref, smem_ref, b256, b0v, sems):
    d1 = pltpu.make_async_copy(
        table_ref.at[pl.ds(0, 1)], smem_ref.at[pl.ds(0, 1)], sems.at[0]
    )
    d2 = pltpu.make_async_copy(
        table_ref.at[pl.ds(256, 1)], smem_ref.at[pl.ds(1, 1)], sems.at[1]
    )
    d1.start()
    d2.start()
    d1.wait()
    d2.wait()

    pending = {0: [], 1: []}
    for h in range(NUM_HEADS):
        p = h % 2
        for d in pending[p]:
            d.wait()
        pending[p] = []
        v256 = smem_ref[1, h]
        v0 = smem_ref[0, h]
        b256[p, ...] = jnp.full((128, T_STATIC), v256, jnp.float32)
        b0v[p, ...] = jnp.full((128, T_STATIC), v0, jnp.float32)
        for g in range(16):
            rows = pl.ds(128 * g, 128)
            wl = 128 * (g - 1)
            if wl > 0:
                d = pltpu.make_async_copy(
                    b256.at[p, :, pl.ds(0, wl)],
                    out_ref.at[0, h, rows, pl.ds(0, wl)],
                    sems.at[p],
                )
                d.start()
                pending[p].append(d)
            wr = 128 * (13 - g)
            if wr > 0:
                d = pltpu.make_async_copy(
                    b0v.at[p, :, pl.ds(0, wr)],
                    out_ref.at[0, h, rows, pl.ds(128 * (g + 3), wr)],
                    sems.at[p],
                )
                d.start()
                pending[p].append(d)
    for p in (0, 1):
        for d in pending[p]:
            d.wait()


@jax.jit
def _run(bias_table):
    tc_const = pl.pallas_call(
        _tc_body,
        out_shape=jax.ShapeDtypeStruct(OUT_SHAPE, jnp.float32),
        in_specs=[pl.BlockSpec(memory_space=pl.ANY)],
        out_specs=pl.BlockSpec(memory_space=pl.ANY),
        scratch_shapes=[
            pltpu.SMEM((2, NUM_HEADS), jnp.float32),
            pltpu.VMEM((2, 128, T_STATIC), jnp.float32),
            pltpu.VMEM((2, 128, T_STATIC), jnp.float32),
            pltpu.SemaphoreType.DMA((2,)),
        ],
    )
    consts = tc_const(bias_table)

    sc_mesh = plsc.VectorSubcoreMesh(
        core_axis_name="c", subcore_axis_name="s", num_cores=2, num_subcores=16
    )
    sc_band = _mpmd._mpmd_map(
        [(sc_mesh, _sc_body)],
        jax.ShapeDtypeStruct(OUT_SHAPE, jnp.float32),
        input_output_aliases={1: 0},
        compiler_params=pltpu.CompilerParams(needs_layout_passes=False),
    )
    return sc_band(bias_table, consts)


def kernel(T, bias_table):
    return _run(bias_table)
